# baseline (speedup 1.0000x reference)
import functools

import jax
import jax.numpy as jnp
from jax import lax
from jax.experimental import pallas as pl
from jax.experimental.pallas import tpu as pltpu
from jax.experimental.pallas import tpu_sc as plsc

NUM_INPUTS = 8192
D_MODEL = 4096
TAU = 1.0

_L = 16
_NS = 16
_CHUNK = NUM_INPUTS // _NS
_NVEC = _CHUNK // _L
_PK = 2 * _L


def _argmax_gather_body(x_hbm, w_hbm, g_hbm, out_hbm,
                        w_v, g_v, stage, stage_idx, sh_val, sh_idx,
                        cval_v, cidx_v, row_v, sem_a, sem_b):
    sid = lax.axis_index("s")
    base = sid * _CHUNK

    cp_w = pltpu.async_copy(w_hbm.at[pl.ds(base, _CHUNK)], w_v, sem_a)
    cp_g = pltpu.async_copy(g_hbm.at[pl.ds(base, _CHUNK)], g_v, sem_b)
    cp_w.wait()
    cp_g.wait()

    lane = lax.iota(jnp.int32, _L)
    neg_inf = jnp.full((_L,), -jnp.inf, jnp.float32)
    bm, bi = neg_inf, jnp.zeros((_L,), jnp.int32)
    for j in range(_NVEC):
        v = w_v[pl.ds(j * _L, _L)] + g_v[pl.ds(j * _L, _L)]
        iv = lane + (base + j * _L)
        m = v > bm
        bm = jnp.where(m, v, bm)
        bi = jnp.where(m, iv, bi)

    stage[...] = bm
    stage_idx[...] = bi
    cp_v = pltpu.async_copy(stage, sh_val.at[pl.ds(sid * _L, _L)], sem_a)
    cp_i = pltpu.async_copy(stage_idx, sh_idx.at[pl.ds(sid * _L, _L)], sem_b)
    cp_v.wait()
    cp_i.wait()
    plsc.subcore_barrier()

    @pl.when(sid == 0)
    def _():
        cp_cv = pltpu.async_copy(sh_val, cval_v, sem_a)
        cp_ci = pltpu.async_copy(sh_idx, cidx_v, sem_b)
        cp_cv.wait()
        cp_ci.wait()
        bm = cval_v[pl.ds(0, _L)]
        bi = cidx_v[pl.ds(0, _L)]
        for i in range(1, _NS):
            v = cval_v[pl.ds(i * _L, _L)]
            iv = cidx_v[pl.ds(i * _L, _L)]
            m = jnp.logical_or(v > bm, jnp.logical_and(v == bm, iv < bi))
            bm = jnp.where(m, v, bm)
            bi = jnp.where(m, iv, bi)
        for sh_amt in (8, 4, 2, 1):
            perm = jnp.bitwise_xor(lane, sh_amt)
            vm = bm.at[perm].get(mode="promise_in_bounds")
            vi = bi.at[perm].get(mode="promise_in_bounds")
            m = jnp.logical_or(vm > bm, jnp.logical_and(vm == bm, vi < bi))
            bm = jnp.where(m, vm, bm)
            bi = jnp.where(m, vi, bi)
        stage_idx[...] = bi
        pltpu.sync_copy(x_hbm.at[stage_idx.at[pl.ds(0, 1)]], row_v)
        pltpu.sync_copy(row_v, out_hbm)


@functools.cache
def _sc_argmax_gather():
    return pl.kernel(
        _argmax_gather_body,
        out_type=jax.ShapeDtypeStruct((1, D_MODEL), jnp.float32),
        mesh=plsc.VectorSubcoreMesh(core_axis_name="c", subcore_axis_name="s",
                                    num_cores=1, num_subcores=16),
        scratch_types=[
            pltpu.VMEM((_CHUNK,), jnp.float32),
            pltpu.VMEM((_CHUNK,), jnp.float32),
            pltpu.VMEM((_L,), jnp.float32),
            pltpu.VMEM((_L,), jnp.int32),
            pltpu.VMEM_SHARED((_NS * _L,), jnp.float32),
            pltpu.VMEM_SHARED((_NS * _L,), jnp.int32),
            pltpu.VMEM((_NS * _L,), jnp.float32),
            pltpu.VMEM((_NS * _L,), jnp.int32),
            pltpu.VMEM((1, D_MODEL), jnp.float32),
            pltpu.SemaphoreType.DMA,
            pltpu.SemaphoreType.DMA,
        ],
    )


def kernel(x, weights):
    gkey = jax.random.key(42)
    u = jax.random.uniform(gkey, weights.shape, dtype=weights.dtype,
                           minval=1e-10, maxval=1.0)
    gumbels = -jnp.log(-jnp.log(u))
    return _sc_argmax_gather()(x, weights, gumbels)


# speedup vs baseline: 5.9765x; 5.9765x over previous
import jax
import jax.numpy as jnp
from jax import lax
from jax.experimental import pallas as pl
from jax.experimental.pallas import tpu as pltpu

NUM_INPUTS = 8192
D_MODEL = 4096
_R = 64
_C = 128


def _body(w_ref, g_ref, x_ref, out_ref, sem):
    v = w_ref[...] + g_ref[...]
    m = jnp.max(v)
    flat = (lax.broadcasted_iota(jnp.int32, (_R, _C), 0) * _C
            + lax.broadcasted_iota(jnp.int32, (_R, _C), 1))
    idx = jnp.min(jnp.where(v == m, flat, 2**31 - 1))
    copy = pltpu.make_async_copy(x_ref.at[pl.ds(idx, 1)], out_ref, sem)
    copy.start()
    copy.wait()


def kernel(x, weights):
    gkey = jax.random.key(42)
    u = jax.random.uniform(gkey, weights.shape, dtype=weights.dtype,
                           minval=1e-10, maxval=1.0)
    gumbels = -jnp.log(-jnp.log(u))
    w2 = weights.reshape(_R, _C)
    g2 = gumbels.reshape(_R, _C)
    return pl.pallas_call(
        _body,
        grid=(1,),
        in_specs=[
            pl.BlockSpec((_R, _C), lambda i: (0, 0)),
            pl.BlockSpec((_R, _C), lambda i: (0, 0)),
            pl.BlockSpec(memory_space=pltpu.MemorySpace.HBM),
        ],
        out_specs=pl.BlockSpec((1, D_MODEL), lambda i: (0, 0)),
        out_shape=jax.ShapeDtypeStruct((1, D_MODEL), jnp.float32),
        scratch_shapes=[pltpu.SemaphoreType.DMA],
    )(w2, g2, x)
